# baseline (speedup 1.0000x reference)
import functools

import jax
import jax.numpy as jnp
from jax import lax
from jax.experimental import pallas as pl
from jax.experimental.pallas import tpu as pltpu
from jax.experimental.pallas import tpu_sc as plsc

_N, _D, _E = 2048, 768, 8
_TILE = 256
_P = 2 * _N + _E * _TILE
_NTILES = _P // _TILE
_NC, _NS = 2, 16
_NW = _NC * _NS
_TOK_W = _N // _NW


def _cv_sq(v):
    eps = 1e-10
    return jnp.var(v, ddof=1) / (jnp.mean(v) ** 2 + eps)


def _route_body(x_ref, wg_ref, p1_ref, p2_ref, g1b_ref, g2b_ref,
                te_ref, tv_ref, stat_ref):
    N = x_ref.shape[0]
    E = wg_ref.shape[1]
    logits = jnp.dot(x_ref[...], wg_ref[...],
                     preferred_element_type=jnp.float32)
    ii = lax.broadcasted_iota(jnp.int32, (N, E), 1)
    l1 = jnp.max(logits, axis=1, keepdims=True)
    i1 = jnp.min(jnp.where(logits >= l1, ii, E), axis=1)
    oh1 = ii == i1[:, None]
    masked = jnp.where(oh1, -1e30, logits)
    l2 = jnp.max(masked, axis=1, keepdims=True)
    i2 = jnp.min(jnp.where(masked >= l2, ii, E), axis=1)
    oh2 = ii == i2[:, None]
    g1v = 1.0 / (1.0 + jnp.exp(l2[:, 0] - l1[:, 0]))
    g2v = 1.0 - g1v

    ohf1 = oh1.astype(jnp.float32)
    ohf2 = oh2.astype(jnp.float32)
    B = 256
    tri = (lax.broadcasted_iota(jnp.int32, (B, B), 0)
           >= lax.broadcasted_iota(jnp.int32, (B, B), 1)).astype(jnp.bfloat16)
    ohb = jnp.concatenate([ohf1, ohf2], axis=0).astype(jnp.bfloat16)
    blocks = []
    run = jnp.zeros((1, E), jnp.float32)
    for b in range(2 * N // B):
        blk = ohb[b * B:(b + 1) * B]
        incl = jnp.dot(tri, blk, preferred_element_type=jnp.float32)
        blocks.append(incl + run)
        run = run + incl[B - 1:B, :]
    inclus = jnp.concatenate(blocks, axis=0)
    ohf = jnp.concatenate([ohf1, ohf2], axis=0)
    c1 = ohf1.sum(axis=0)
    counts = c1 + ohf2.sum(axis=0)
    rank12 = inclus - ohf
    rank1 = rank12[:N]
    rank2 = rank12[N:]
    padded = jnp.ceil(counts / _TILE) * _TILE
    le = (lax.broadcasted_iota(jnp.int32, (E, E), 0)
          <= lax.broadcasted_iota(jnp.int32, (E, E), 1)).astype(jnp.float32)
    pad_end = jnp.sum(padded[:, None] * le, axis=0)
    pad_start = pad_end - padded
    slot1 = (ohf1 * (rank1 + pad_start[None, :])).sum(axis=1)
    slot2 = (ohf2 * (rank2 + pad_start[None, :])).sum(axis=1)
    p1_ref[...] = slot1.astype(jnp.int32)
    p2_ref[...] = slot2.astype(jnp.int32)
    g1b_ref[...] = jnp.broadcast_to(g1v[:, None], (N, 16))
    g2b_ref[...] = jnp.broadcast_to(g2v[:, None], (N, 16))

    ts = (lax.broadcasted_iota(jnp.int32, (1, 128), 1)
          .astype(jnp.float32) * _TILE)
    ge = ts >= pad_end[:, None]
    tile_e = ge.astype(jnp.int32).sum(axis=0)
    tile_e_c = jnp.minimum(tile_e, E - 1)
    oht = (lax.broadcasted_iota(jnp.int32, (E, 128), 0)
           == tile_e_c[None, :]).astype(jnp.float32)
    ps_t = (oht * pad_start[:, None]).sum(axis=0)
    cnt_t = (oht * counts[:, None]).sum(axis=0)
    tile_valid = ((tile_e < E)
                  & ((ts[0] - ps_t) < cnt_t)).astype(jnp.int32)
    te_ref[...] = tile_e_c
    tv_ref[...] = tile_valid

    imp = (ohf1 * g1v[:, None] + ohf2 * g2v[:, None]).sum(axis=0)
    stat_ref[0, 0:E] = imp
    stat_ref[0, E:2 * E] = counts


def _group_body(te_ref, tv_ref, x_ref, w1_ref, b1_ref, w2_ref, b2_ref,
                o_ref):
    t = pl.program_id(0)

    @pl.when(tv_ref[t] != 0)
    def _():
        xb = x_ref[...]
        h = jnp.dot(xb, w1_ref[0], preferred_element_type=jnp.float32)
        h = jnp.maximum(h + b1_ref[0], 0.0).astype(jnp.bfloat16)
        y = jnp.dot(h, w2_ref[0], preferred_element_type=jnp.float32)
        o_ref[...] = y + b2_ref[0]


def _sc_dispatch_body(x_hbm, p1_hbm, p2_hbm, out_hbm, i1_v, i2_v, rows_v,
                      sem):
    wid = lax.axis_index("s") * _NC + lax.axis_index("c")
    base = wid * _TOK_W
    a1 = pltpu.async_copy(p1_hbm.at[pl.ds(base, _TOK_W)], i1_v, sem)
    a2 = pltpu.async_copy(p2_hbm.at[pl.ds(base, _TOK_W)], i2_v, sem)
    a3 = pltpu.async_copy(x_hbm.at[pl.ds(base, _TOK_W)], rows_v, sem)
    a1.wait()
    a2.wait()
    a3.wait()
    c1 = pltpu.async_copy(rows_v, out_hbm.at[i1_v], sem)
    c2 = pltpu.async_copy(rows_v, out_hbm.at[i2_v], sem)
    c1.wait()
    c2.wait()


def _sc_combine_body(y_hbm, p1_hbm, p2_hbm, g1_hbm, g2_hbm, out_hbm,
                     i1_v, i2_v, g1_v, g2_v, r1_v, r2_v, sem):
    wid = lax.axis_index("s") * _NC + lax.axis_index("c")
    base = wid * _TOK_W
    a1 = pltpu.async_copy(p1_hbm.at[pl.ds(base, _TOK_W)], i1_v, sem)
    a2 = pltpu.async_copy(p2_hbm.at[pl.ds(base, _TOK_W)], i2_v, sem)
    a3 = pltpu.async_copy(g1_hbm.at[pl.ds(base, _TOK_W)], g1_v, sem)
    a4 = pltpu.async_copy(g2_hbm.at[pl.ds(base, _TOK_W)], g2_v, sem)
    a1.wait()
    a2.wait()
    a3.wait()
    a4.wait()
    c1 = pltpu.async_copy(y_hbm.at[i1_v], r1_v, sem)
    c2 = pltpu.async_copy(y_hbm.at[i2_v], r2_v, sem)
    c1.wait()
    c2.wait()

    def row_fma(i, carry):
        ga = g1_v[i, pl.ds(0, 16)]
        gb = g2_v[i, pl.ds(0, 16)]
        for j in range(_D // 16):
            s = pl.ds(j * 16, 16)
            r1_v[i, s] = r1_v[i, s] * ga + r2_v[i, s] * gb
        return carry

    lax.fori_loop(0, _TOK_W, row_fma, 0)
    pltpu.sync_copy(r1_v, out_hbm.at[pl.ds(base, _TOK_W)])


@functools.cache
def _sc_kernels():
    mesh = plsc.VectorSubcoreMesh(core_axis_name="c", subcore_axis_name="s")
    dispatch = pl.kernel(
        _sc_dispatch_body,
        out_type=jax.ShapeDtypeStruct((_P, _D // 2), jnp.int32),
        mesh=mesh,
        scratch_types=[
            pltpu.VMEM((_TOK_W,), jnp.int32),
            pltpu.VMEM((_TOK_W,), jnp.int32),
            pltpu.VMEM((_TOK_W, _D // 2), jnp.int32),
            pltpu.SemaphoreType.DMA,
        ],
    )
    combine = pl.kernel(
        _sc_combine_body,
        out_type=jax.ShapeDtypeStruct((_N, _D), jnp.float32),
        mesh=mesh,
        scratch_types=[
            pltpu.VMEM((_TOK_W,), jnp.int32),
            pltpu.VMEM((_TOK_W,), jnp.int32),
            pltpu.VMEM((_TOK_W, 16), jnp.float32),
            pltpu.VMEM((_TOK_W, 16), jnp.float32),
            pltpu.VMEM((_TOK_W, _D), jnp.float32),
            pltpu.VMEM((_TOK_W, _D), jnp.float32),
            pltpu.SemaphoreType.DMA,
        ],
    )
    return dispatch, combine


def kernel(x, w_gate, W1, b1, W2, b2):
    N, D = x.shape
    E = w_gate.shape[1]
    H = W1.shape[2]

    p1, p2, g1b, g2b, tile_e_c, tile_valid, stats = pl.pallas_call(
        _route_body,
        out_shape=[
            jax.ShapeDtypeStruct((N,), jnp.int32),
            jax.ShapeDtypeStruct((N,), jnp.int32),
            jax.ShapeDtypeStruct((N, 16), jnp.float32),
            jax.ShapeDtypeStruct((N, 16), jnp.float32),
            jax.ShapeDtypeStruct((128,), jnp.int32),
            jax.ShapeDtypeStruct((128,), jnp.int32),
            jax.ShapeDtypeStruct((1, 128), jnp.float32),
        ],
    )(x, w_gate)
    importance = stats[0, 0:E]
    load = stats[0, E:2 * E]
    loss = (_cv_sq(importance) + _cv_sq(load)) * 1e-2

    _sc_dispatch, _sc_combine = _sc_kernels()
    x_packed = lax.bitcast_convert_type(
        x.astype(jnp.bfloat16).reshape(N, D // 2, 2), jnp.int32)
    x_sorted = lax.bitcast_convert_type(
        _sc_dispatch(x_packed, p1, p2), jnp.bfloat16).reshape(_P, D)

    grid_spec = pltpu.PrefetchScalarGridSpec(
        num_scalar_prefetch=2,
        grid=(_NTILES,),
        in_specs=[
            pl.BlockSpec((_TILE, D), lambda t, te, tv: (t, 0)),
            pl.BlockSpec((1, D, H), lambda t, te, tv: (te[t], 0, 0)),
            pl.BlockSpec((1, 1, H), lambda t, te, tv: (te[t], 0, 0)),
            pl.BlockSpec((1, H, D), lambda t, te, tv: (te[t], 0, 0)),
            pl.BlockSpec((1, 1, D), lambda t, te, tv: (te[t], 0, 0)),
        ],
        out_specs=pl.BlockSpec((_TILE, D), lambda t, te, tv: (t, 0)),
    )
    y_sorted = pl.pallas_call(
        _group_body,
        grid_spec=grid_spec,
        out_shape=jax.ShapeDtypeStruct((_P, D), jnp.float32),
        compiler_params=pltpu.CompilerParams(
            dimension_semantics=("arbitrary",)),
    )(tile_e_c, tile_valid, x_sorted, W1.astype(jnp.bfloat16),
      b1.reshape(E, 1, H), W2.astype(jnp.bfloat16), b2.reshape(E, 1, D))

    y = _sc_combine(y_sorted, p1, p2, g1b, g2b)
    return y, loss


# speedup vs baseline: 2.2423x; 2.2423x over previous
import functools

import jax
import jax.numpy as jnp
from jax import lax
from jax.experimental import pallas as pl
from jax.experimental.pallas import tpu as pltpu
from jax.experimental.pallas import tpu_sc as plsc

_N, _D, _E = 2048, 768, 8
_TILE = 256
_P = 2 * _N + _E * _TILE
_NTILES = _P // _TILE
_NC, _NS = 2, 16
_NW = _NC * _NS
_TOK_W = _N // _NW


def _cv_sq(v):
    eps = 1e-10
    return jnp.var(v, ddof=1) / (jnp.mean(v) ** 2 + eps)


def _route_body(x_ref, wg_ref, p1_ref, p2_ref, g1b_ref, g2b_ref,
                te_ref, tv_ref, stat_ref):
    N = x_ref.shape[0]
    E = wg_ref.shape[1]
    logits = jnp.dot(x_ref[...], wg_ref[...],
                     preferred_element_type=jnp.float32)
    ii = lax.broadcasted_iota(jnp.int32, (N, E), 1)
    l1 = jnp.max(logits, axis=1, keepdims=True)
    i1 = jnp.min(jnp.where(logits >= l1, ii, E), axis=1)
    oh1 = ii == i1[:, None]
    masked = jnp.where(oh1, -1e30, logits)
    l2 = jnp.max(masked, axis=1, keepdims=True)
    i2 = jnp.min(jnp.where(masked >= l2, ii, E), axis=1)
    oh2 = ii == i2[:, None]
    g1v = 1.0 / (1.0 + jnp.exp(l2[:, 0] - l1[:, 0]))
    g2v = 1.0 - g1v

    ohf1 = oh1.astype(jnp.float32)
    ohf2 = oh2.astype(jnp.float32)
    B = 256
    tri = (lax.broadcasted_iota(jnp.int32, (B, B), 0)
           >= lax.broadcasted_iota(jnp.int32, (B, B), 1)).astype(jnp.bfloat16)
    ohb = jnp.concatenate([ohf1, ohf2], axis=0).astype(jnp.bfloat16)
    blocks = []
    run = jnp.zeros((1, E), jnp.float32)
    for b in range(2 * N // B):
        blk = ohb[b * B:(b + 1) * B]
        incl = jnp.dot(tri, blk, preferred_element_type=jnp.float32)
        blocks.append(incl + run)
        run = run + incl[B - 1:B, :]
    inclus = jnp.concatenate(blocks, axis=0)
    ohf = jnp.concatenate([ohf1, ohf2], axis=0)
    c1 = ohf1.sum(axis=0)
    counts = c1 + ohf2.sum(axis=0)
    rank12 = inclus - ohf
    rank1 = rank12[:N]
    rank2 = rank12[N:]
    padded = jnp.ceil(counts / _TILE) * _TILE
    le = (lax.broadcasted_iota(jnp.int32, (E, E), 0)
          <= lax.broadcasted_iota(jnp.int32, (E, E), 1)).astype(jnp.float32)
    pad_end = jnp.sum(padded[:, None] * le, axis=0)
    pad_start = pad_end - padded
    slot1 = (ohf1 * (rank1 + pad_start[None, :])).sum(axis=1)
    slot2 = (ohf2 * (rank2 + pad_start[None, :])).sum(axis=1)
    p1_ref[...] = slot1.astype(jnp.int32)
    p2_ref[...] = slot2.astype(jnp.int32)
    g1b_ref[...] = jnp.broadcast_to(g1v[:, None], (N, 16))
    g2b_ref[...] = jnp.broadcast_to(g2v[:, None], (N, 16))

    ts = (lax.broadcasted_iota(jnp.int32, (1, 128), 1)
          .astype(jnp.float32) * _TILE)
    ge = ts >= pad_end[:, None]
    tile_e = ge.astype(jnp.int32).sum(axis=0)
    tile_e_c = jnp.minimum(tile_e, E - 1)
    oht = (lax.broadcasted_iota(jnp.int32, (E, 128), 0)
           == tile_e_c[None, :]).astype(jnp.float32)
    ps_t = (oht * pad_start[:, None]).sum(axis=0)
    cnt_t = (oht * counts[:, None]).sum(axis=0)
    tile_valid = ((tile_e < E)
                  & ((ts[0] - ps_t) < cnt_t)).astype(jnp.int32)
    te_ref[...] = tile_e_c
    tv_ref[...] = tile_valid

    imp = (ohf1 * g1v[:, None] + ohf2 * g2v[:, None]).sum(axis=0)
    stat_ref[0, 0:E] = imp
    stat_ref[0, E:2 * E] = counts


def _group_body(te_ref, tv_ref, x_ref, w1_ref, b1_ref, w2_ref, b2_ref,
                o_ref):
    t = pl.program_id(0)

    @pl.when(tv_ref[t] != 0)
    def _():
        xb = x_ref[...].astype(jnp.bfloat16)
        h = jnp.dot(xb, w1_ref[0], preferred_element_type=jnp.float32)
        h = jnp.maximum(h + b1_ref[0], 0.0).astype(jnp.bfloat16)
        y = jnp.dot(h, w2_ref[0], preferred_element_type=jnp.float32)
        o_ref[...] = y + b2_ref[0]


def _sc_dispatch_body(x_hbm, p1_hbm, p2_hbm, out_hbm, i1_v, i2_v, rows_v,
                      sem):
    wid = lax.axis_index("s") * _NC + lax.axis_index("c")
    base = wid * _TOK_W
    a1 = pltpu.async_copy(p1_hbm.at[pl.ds(base, _TOK_W)], i1_v, sem)
    a2 = pltpu.async_copy(p2_hbm.at[pl.ds(base, _TOK_W)], i2_v, sem)
    a3 = pltpu.async_copy(x_hbm.at[pl.ds(base, _TOK_W)], rows_v, sem)
    a1.wait()
    a2.wait()
    a3.wait()
    c1 = pltpu.async_copy(rows_v, out_hbm.at[i1_v], sem)
    c2 = pltpu.async_copy(rows_v, out_hbm.at[i2_v], sem)
    c1.wait()
    c2.wait()


def _sc_combine_body(y_hbm, p1_hbm, p2_hbm, g1_hbm, g2_hbm, out_hbm,
                     i1_v, i2_v, g1_v, g2_v, r1_v, r2_v, sem):
    wid = lax.axis_index("s") * _NC + lax.axis_index("c")
    base = wid * _TOK_W
    a1 = pltpu.async_copy(p1_hbm.at[pl.ds(base, _TOK_W)], i1_v, sem)
    a2 = pltpu.async_copy(p2_hbm.at[pl.ds(base, _TOK_W)], i2_v, sem)
    a3 = pltpu.async_copy(g1_hbm.at[pl.ds(base, _TOK_W)], g1_v, sem)
    a4 = pltpu.async_copy(g2_hbm.at[pl.ds(base, _TOK_W)], g2_v, sem)
    a1.wait()
    a2.wait()
    a3.wait()
    a4.wait()
    c1 = pltpu.async_copy(y_hbm.at[i1_v], r1_v, sem)
    c2 = pltpu.async_copy(y_hbm.at[i2_v], r2_v, sem)
    c1.wait()
    c2.wait()

    def row_fma(i, carry):
        ga = g1_v[i, pl.ds(0, 16)]
        gb = g2_v[i, pl.ds(0, 16)]
        for j in range(_D // 16):
            s = pl.ds(j * 16, 16)
            r1_v[i, s] = r1_v[i, s] * ga + r2_v[i, s] * gb
        return carry

    lax.fori_loop(0, _TOK_W, row_fma, 0)
    pltpu.sync_copy(r1_v, out_hbm.at[pl.ds(base, _TOK_W)])


@functools.cache
def _sc_kernels():
    mesh = plsc.VectorSubcoreMesh(core_axis_name="c", subcore_axis_name="s")
    dispatch = pl.kernel(
        _sc_dispatch_body,
        out_type=jax.ShapeDtypeStruct((_P, _D), jnp.float32),
        mesh=mesh,
        scratch_types=[
            pltpu.VMEM((_TOK_W,), jnp.int32),
            pltpu.VMEM((_TOK_W,), jnp.int32),
            pltpu.VMEM((_TOK_W, _D), jnp.float32),
            pltpu.SemaphoreType.DMA,
        ],
    )
    combine = pl.kernel(
        _sc_combine_body,
        out_type=jax.ShapeDtypeStruct((_N, _D), jnp.float32),
        mesh=mesh,
        scratch_types=[
            pltpu.VMEM((_TOK_W,), jnp.int32),
            pltpu.VMEM((_TOK_W,), jnp.int32),
            pltpu.VMEM((_TOK_W, 16), jnp.float32),
            pltpu.VMEM((_TOK_W, 16), jnp.float32),
            pltpu.VMEM((_TOK_W, _D), jnp.float32),
            pltpu.VMEM((_TOK_W, _D), jnp.float32),
            pltpu.SemaphoreType.DMA,
        ],
    )
    return dispatch, combine


def kernel(x, w_gate, W1, b1, W2, b2):
    N, D = x.shape
    E = w_gate.shape[1]
    H = W1.shape[2]

    p1, p2, g1b, g2b, tile_e_c, tile_valid, stats = pl.pallas_call(
        _route_body,
        out_shape=[
            jax.ShapeDtypeStruct((N,), jnp.int32),
            jax.ShapeDtypeStruct((N,), jnp.int32),
            jax.ShapeDtypeStruct((N, 16), jnp.float32),
            jax.ShapeDtypeStruct((N, 16), jnp.float32),
            jax.ShapeDtypeStruct((128,), jnp.int32),
            jax.ShapeDtypeStruct((128,), jnp.int32),
            jax.ShapeDtypeStruct((1, 128), jnp.float32),
        ],
    )(x, w_gate)
    importance = stats[0, 0:E]
    load = stats[0, E:2 * E]
    loss = (_cv_sq(importance) + _cv_sq(load)) * 1e-2

    _sc_dispatch, _sc_combine = _sc_kernels()
    x_sorted = _sc_dispatch(x, p1, p2)

    grid_spec = pltpu.PrefetchScalarGridSpec(
        num_scalar_prefetch=2,
        grid=(_NTILES,),
        in_specs=[
            pl.BlockSpec((_TILE, D), lambda t, te, tv: (t, 0)),
            pl.BlockSpec((1, D, H), lambda t, te, tv: (te[t], 0, 0)),
            pl.BlockSpec((1, 1, H), lambda t, te, tv: (te[t], 0, 0)),
            pl.BlockSpec((1, H, D), lambda t, te, tv: (te[t], 0, 0)),
            pl.BlockSpec((1, 1, D), lambda t, te, tv: (te[t], 0, 0)),
        ],
        out_specs=pl.BlockSpec((_TILE, D), lambda t, te, tv: (t, 0)),
    )
    y_sorted = pl.pallas_call(
        _group_body,
        grid_spec=grid_spec,
        out_shape=jax.ShapeDtypeStruct((_P, D), jnp.float32),
        compiler_params=pltpu.CompilerParams(
            dimension_semantics=("arbitrary",)),
    )(tile_e_c, tile_valid, x_sorted, W1.astype(jnp.bfloat16),
      b1.reshape(E, 1, H), W2.astype(jnp.bfloat16), b2.reshape(E, 1, D))

    y = _sc_combine(y_sorted, p1, p2, g1b, g2b)
    return y, loss


# speedup vs baseline: 2.3715x; 1.0576x over previous
import functools

import jax
import jax.numpy as jnp
from jax import lax
from jax.experimental import pallas as pl
from jax.experimental.pallas import tpu as pltpu
from jax.experimental.pallas import tpu_sc as plsc

_N, _D, _E = 2048, 768, 8
_TILE = 512
_P = 2 * _N + _E * _TILE
_NTILES = _P // _TILE
_NC, _NS = 2, 16
_NW = _NC * _NS
_TOK_W = _N // _NW


def _cv_sq(v):
    eps = 1e-10
    return jnp.var(v, ddof=1) / (jnp.mean(v) ** 2 + eps)


def _route_body(x_ref, wg_ref, p1_ref, p2_ref, g1b_ref, g2b_ref,
                te_ref, tv_ref, stat_ref):
    N = x_ref.shape[0]
    E = wg_ref.shape[1]
    logits = jnp.dot(x_ref[...], wg_ref[...],
                     preferred_element_type=jnp.float32)
    ii = lax.broadcasted_iota(jnp.int32, (N, E), 1)
    l1 = jnp.max(logits, axis=1, keepdims=True)
    i1 = jnp.min(jnp.where(logits >= l1, ii, E), axis=1)
    oh1 = ii == i1[:, None]
    masked = jnp.where(oh1, -1e30, logits)
    l2 = jnp.max(masked, axis=1, keepdims=True)
    i2 = jnp.min(jnp.where(masked >= l2, ii, E), axis=1)
    oh2 = ii == i2[:, None]
    g1v = 1.0 / (1.0 + jnp.exp(l2[:, 0] - l1[:, 0]))
    g2v = 1.0 - g1v

    ohf1 = oh1.astype(jnp.float32)
    ohf2 = oh2.astype(jnp.float32)
    B = 256
    tri = (lax.broadcasted_iota(jnp.int32, (B, B), 0)
           >= lax.broadcasted_iota(jnp.int32, (B, B), 1)).astype(jnp.bfloat16)
    ohb = jnp.concatenate([ohf1, ohf2], axis=0).astype(jnp.bfloat16)
    blocks = []
    run = jnp.zeros((1, E), jnp.float32)
    for b in range(2 * N // B):
        blk = ohb[b * B:(b + 1) * B]
        incl = jnp.dot(tri, blk, preferred_element_type=jnp.float32)
        blocks.append(incl + run)
        run = run + incl[B - 1:B, :]
    inclus = jnp.concatenate(blocks, axis=0)
    ohf = jnp.concatenate([ohf1, ohf2], axis=0)
    c1 = ohf1.sum(axis=0)
    counts = c1 + ohf2.sum(axis=0)
    rank12 = inclus - ohf
    rank1 = rank12[:N]
    rank2 = rank12[N:]
    padded = jnp.ceil(counts / _TILE) * _TILE
    le = (lax.broadcasted_iota(jnp.int32, (E, E), 0)
          <= lax.broadcasted_iota(jnp.int32, (E, E), 1)).astype(jnp.float32)
    pad_end = jnp.sum(padded[:, None] * le, axis=0)
    pad_start = pad_end - padded
    slot1 = (ohf1 * (rank1 + pad_start[None, :])).sum(axis=1)
    slot2 = (ohf2 * (rank2 + pad_start[None, :])).sum(axis=1)
    p1_ref[...] = slot1.astype(jnp.int32)
    p2_ref[...] = slot2.astype(jnp.int32)
    g1b_ref[...] = jnp.broadcast_to(g1v[:, None], (N, 16))
    g2b_ref[...] = jnp.broadcast_to(g2v[:, None], (N, 16))

    ts = (lax.broadcasted_iota(jnp.int32, (1, 128), 1)
          .astype(jnp.float32) * _TILE)
    ge = ts >= pad_end[:, None]
    tile_e = ge.astype(jnp.int32).sum(axis=0)
    tile_e_c = jnp.minimum(tile_e, E - 1)
    oht = (lax.broadcasted_iota(jnp.int32, (E, 128), 0)
           == tile_e_c[None, :]).astype(jnp.float32)
    ps_t = (oht * pad_start[:, None]).sum(axis=0)
    cnt_t = (oht * counts[:, None]).sum(axis=0)
    tile_valid = ((tile_e < E)
                  & ((ts[0] - ps_t) < cnt_t)).astype(jnp.int32)
    te_ref[...] = tile_e_c
    tv_ref[...] = tile_valid

    imp = (ohf1 * g1v[:, None] + ohf2 * g2v[:, None]).sum(axis=0)

    def cv2(v):
        m = jnp.sum(v) / E
        var = jnp.sum((v - m) ** 2) / (E - 1)
        return var / (m * m + 1e-10)

    loss = (cv2(imp) + cv2(counts)) * 1e-2
    stat_ref[...] = jnp.full((1, 128), loss, jnp.float32)


def _group_body(te_ref, tv_ref, x_ref, w1_ref, b1_ref, w2_ref, b2_ref,
                o_ref):
    t = pl.program_id(0)

    @pl.when(tv_ref[t] != 0)
    def _():
        xb = x_ref[...].astype(jnp.bfloat16)
        h = jnp.dot(xb, w1_ref[0], preferred_element_type=jnp.float32)
        h = jnp.maximum(h + b1_ref[0], 0.0).astype(jnp.bfloat16)
        y = jnp.dot(h, w2_ref[0], preferred_element_type=jnp.float32)
        o_ref[...] = y + b2_ref[0]


def _sc_dispatch_body(x_hbm, p1_hbm, p2_hbm, out_hbm, i1_v, i2_v, rows_v,
                      sem):
    wid = lax.axis_index("s") * _NC + lax.axis_index("c")
    base = wid * _TOK_W
    a1 = pltpu.async_copy(p1_hbm.at[pl.ds(base, _TOK_W)], i1_v, sem)
    a2 = pltpu.async_copy(p2_hbm.at[pl.ds(base, _TOK_W)], i2_v, sem)
    a3 = pltpu.async_copy(x_hbm.at[pl.ds(base, _TOK_W)], rows_v, sem)
    a1.wait()
    a2.wait()
    a3.wait()
    c1 = pltpu.async_copy(rows_v, out_hbm.at[i1_v], sem)
    c2 = pltpu.async_copy(rows_v, out_hbm.at[i2_v], sem)
    c1.wait()
    c2.wait()


def _sc_combine_body(y_hbm, p1_hbm, p2_hbm, g1_hbm, g2_hbm, out_hbm,
                     i1_v, i2_v, g1_v, g2_v, r1_v, r2_v, sem):
    wid = lax.axis_index("s") * _NC + lax.axis_index("c")
    base = wid * _TOK_W
    a1 = pltpu.async_copy(p1_hbm.at[pl.ds(base, _TOK_W)], i1_v, sem)
    a2 = pltpu.async_copy(p2_hbm.at[pl.ds(base, _TOK_W)], i2_v, sem)
    a3 = pltpu.async_copy(g1_hbm.at[pl.ds(base, _TOK_W)], g1_v, sem)
    a4 = pltpu.async_copy(g2_hbm.at[pl.ds(base, _TOK_W)], g2_v, sem)
    a1.wait()
    a2.wait()
    a3.wait()
    a4.wait()
    c1 = pltpu.async_copy(y_hbm.at[i1_v], r1_v, sem)
    c2 = pltpu.async_copy(y_hbm.at[i2_v], r2_v, sem)
    c1.wait()
    c2.wait()

    def row_fma(i, carry):
        ga = g1_v[i, pl.ds(0, 16)]
        gb = g2_v[i, pl.ds(0, 16)]
        for j in range(_D // 16):
            s = pl.ds(j * 16, 16)
            r1_v[i, s] = r1_v[i, s] * ga + r2_v[i, s] * gb
        return carry

    lax.fori_loop(0, _TOK_W, row_fma, 0)
    pltpu.sync_copy(r1_v, out_hbm.at[pl.ds(base, _TOK_W)])


@functools.cache
def _sc_kernels():
    mesh = plsc.VectorSubcoreMesh(core_axis_name="c", subcore_axis_name="s")
    dispatch = pl.kernel(
        _sc_dispatch_body,
        out_type=jax.ShapeDtypeStruct((_P, _D), jnp.float32),
        mesh=mesh,
        scratch_types=[
            pltpu.VMEM((_TOK_W,), jnp.int32),
            pltpu.VMEM((_TOK_W,), jnp.int32),
            pltpu.VMEM((_TOK_W, _D), jnp.float32),
            pltpu.SemaphoreType.DMA,
        ],
    )
    combine = pl.kernel(
        _sc_combine_body,
        out_type=jax.ShapeDtypeStruct((_N, _D), jnp.float32),
        mesh=mesh,
        scratch_types=[
            pltpu.VMEM((_TOK_W,), jnp.int32),
            pltpu.VMEM((_TOK_W,), jnp.int32),
            pltpu.VMEM((_TOK_W, 16), jnp.float32),
            pltpu.VMEM((_TOK_W, 16), jnp.float32),
            pltpu.VMEM((_TOK_W, _D), jnp.float32),
            pltpu.VMEM((_TOK_W, _D), jnp.float32),
            pltpu.SemaphoreType.DMA,
        ],
    )
    return dispatch, combine


def kernel(x, w_gate, W1, b1, W2, b2):
    N, D = x.shape
    E = w_gate.shape[1]
    H = W1.shape[2]

    p1, p2, g1b, g2b, tile_e_c, tile_valid, stats = pl.pallas_call(
        _route_body,
        out_shape=[
            jax.ShapeDtypeStruct((N,), jnp.int32),
            jax.ShapeDtypeStruct((N,), jnp.int32),
            jax.ShapeDtypeStruct((N, 16), jnp.float32),
            jax.ShapeDtypeStruct((N, 16), jnp.float32),
            jax.ShapeDtypeStruct((128,), jnp.int32),
            jax.ShapeDtypeStruct((128,), jnp.int32),
            jax.ShapeDtypeStruct((1, 128), jnp.float32),
        ],
    )(x, w_gate)
    loss = stats[0, 0]

    _sc_dispatch, _sc_combine = _sc_kernels()
    x_sorted = _sc_dispatch(x, p1, p2)

    grid_spec = pltpu.PrefetchScalarGridSpec(
        num_scalar_prefetch=2,
        grid=(_NTILES,),
        in_specs=[
            pl.BlockSpec((_TILE, D), lambda t, te, tv: (t, 0)),
            pl.BlockSpec((1, D, H), lambda t, te, tv: (te[t], 0, 0)),
            pl.BlockSpec((1, 1, H), lambda t, te, tv: (te[t], 0, 0)),
            pl.BlockSpec((1, H, D), lambda t, te, tv: (te[t], 0, 0)),
            pl.BlockSpec((1, 1, D), lambda t, te, tv: (te[t], 0, 0)),
        ],
        out_specs=pl.BlockSpec((_TILE, D), lambda t, te, tv: (t, 0)),
    )
    y_sorted = pl.pallas_call(
        _group_body,
        grid_spec=grid_spec,
        out_shape=jax.ShapeDtypeStruct((_P, D), jnp.float32),
        compiler_params=pltpu.CompilerParams(
            dimension_semantics=("arbitrary",)),
    )(tile_e_c, tile_valid, x_sorted, W1.astype(jnp.bfloat16),
      b1.reshape(E, 1, H), W2.astype(jnp.bfloat16), b2.reshape(E, 1, D))

    y = _sc_combine(y_sorted, p1, p2, g1b, g2b)
    return y, loss
